# Initial kernel scaffold; baseline (speedup 1.0000x reference)
import jax
import jax.numpy as jnp
from jax.experimental import pallas as pl


def kernel(token_ids, weight_float):
    raise NotImplementedError("write your pallas kernel here")


# speedup vs baseline: 11.4045x; 11.4045x over previous
import functools

import jax
import jax.numpy as jnp
from jax import lax
from jax.experimental import pallas as pl
from jax.experimental.pallas import tpu as pltpu
from jax.experimental.pallas import tpu_sc as plsc

_B = 1024
_D = 16
_NBITS = 32


def _spike_embed_call(token_ids, weight_float):
    info = plsc.get_sparse_core_info()
    nc, ns, nl = info.num_cores, info.num_subcores, info.num_lanes
    nw = nc * ns
    bpw = _B // nw

    mesh = plsc.VectorSubcoreMesh(core_axis_name="c", subcore_axis_name="s")

    @functools.partial(
        pl.kernel,
        mesh=mesh,
        out_type=jax.ShapeDtypeStruct((_B, _D * _NBITS), jnp.float32),
        scratch_types=[
            pltpu.VMEM((bpw,), jnp.int32),
            pltpu.VMEM((bpw, _D), jnp.int32),
            pltpu.VMEM((bpw, _D * _NBITS), jnp.float32),
            pltpu.SemaphoreType.DMA,
        ],
        compiler_params=pltpu.CompilerParams(
            needs_layout_passes=False, use_tc_tiling_on_sc=False),
    )
    def spike_embed(ids_hbm, table_hbm, out_hbm, idx_v, rows_v, out_v, sem):
        wid = lax.axis_index("s") * nc + lax.axis_index("c")
        base = wid * bpw
        pltpu.sync_copy(ids_hbm.at[pl.ds(base, bpw)], idx_v)
        pltpu.async_copy(table_hbm.at[idx_v], rows_v, sem).wait()

        lane = lax.iota(jnp.int32, nl)
        lane_stride = lane * _NBITS
        zeros = jnp.zeros((nl,), jnp.int32)

        def token_body(t, carry):
            row = rows_v[t]
            t_vec = zeros + t
            for k in range(_NBITS):
                bits = ((row >> (31 - k)) & 1).astype(jnp.float32)
                plsc.store_scatter(out_v, [t_vec, lane_stride + k], bits)
            return carry

        lax.fori_loop(0, bpw, token_body, 0)
        pltpu.sync_copy(out_v, out_hbm.at[pl.ds(base, bpw)])

    return spike_embed(token_ids, weight_float)


def kernel(token_ids, weight_float):
    table_bits = jax.lax.bitcast_convert_type(
        weight_float.astype(jnp.float32), jnp.int32)
    out = _spike_embed_call(token_ids.astype(jnp.int32), table_bits)
    return out.reshape(_B, _D, _NBITS)


# speedup vs baseline: 14.3057x; 1.2544x over previous
import functools

import jax
import jax.numpy as jnp
from jax import lax
from jax.experimental import pallas as pl
from jax.experimental.pallas import tpu as pltpu
from jax.experimental.pallas import tpu_sc as plsc

_B = 1024
_D = 16
_NBITS = 32


def _spike_embed_call(token_ids, weight_float):
    info = plsc.get_sparse_core_info()
    nc, ns, nl = info.num_cores, info.num_subcores, info.num_lanes
    nw = nc * ns
    bpw = _B // nw

    mesh = plsc.VectorSubcoreMesh(core_axis_name="c", subcore_axis_name="s")

    @functools.partial(
        pl.kernel,
        mesh=mesh,
        out_type=jax.ShapeDtypeStruct((_B, _D * _NBITS), jnp.float32),
        scratch_types=[
            pltpu.VMEM((bpw,), jnp.int32),
            pltpu.VMEM((bpw, _D), jnp.int32),
            pltpu.VMEM((bpw, _D * _NBITS), jnp.float32),
            pltpu.SemaphoreType.DMA,
        ],
        compiler_params=pltpu.CompilerParams(
            needs_layout_passes=False, use_tc_tiling_on_sc=False),
    )
    def spike_embed(ids_hbm, table_hbm, out_hbm, idx_v, rows_v, out_v, sem):
        wid = lax.axis_index("s") * nc + lax.axis_index("c")
        base = wid * bpw
        pltpu.sync_copy(ids_hbm.at[pl.ds(base, bpw)], idx_v)
        pltpu.async_copy(table_hbm.at[idx_v], rows_v, sem).wait()

        lane = lax.iota(jnp.int32, nl)
        shifts = [31 - lane, 15 - lane]
        zeros = jnp.zeros((nl,), jnp.int32)

        def token_body(t, carry):
            row = rows_v[t]
            for d in range(_D):
                word = zeros + row[d]
                for h in range(2):
                    bits = ((word >> shifts[h]) & 1).astype(jnp.float32)
                    out_v[t, pl.ds(d * _NBITS + h * nl, nl)] = bits
            return carry

        lax.fori_loop(0, bpw, token_body, 0)
        pltpu.sync_copy(out_v, out_hbm.at[pl.ds(base, bpw)])

    return spike_embed(token_ids, weight_float)


def kernel(token_ids, weight_float):
    table_bits = jax.lax.bitcast_convert_type(
        weight_float.astype(jnp.float32), jnp.int32)
    out = _spike_embed_call(token_ids.astype(jnp.int32), table_bits)
    return out.reshape(_B, _D, _NBITS)


# speedup vs baseline: 14.3516x; 1.0032x over previous
import functools

import jax
import jax.numpy as jnp
from jax import lax
from jax.experimental import pallas as pl
from jax.experimental.pallas import tpu as pltpu
from jax.experimental.pallas import tpu_sc as plsc

_B = 1024
_D = 16
_NBITS = 32


def _spike_embed_call(token_ids, weight_float):
    info = plsc.get_sparse_core_info()
    nc, ns, nl = info.num_cores, info.num_subcores, info.num_lanes
    nw = nc * ns
    bpw = _B // nw

    mesh = plsc.VectorSubcoreMesh(core_axis_name="c", subcore_axis_name="s")

    @functools.partial(
        pl.kernel,
        mesh=mesh,
        out_type=jax.ShapeDtypeStruct((_B, _D * _NBITS), jnp.float32),
        scratch_types=[
            pltpu.VMEM((bpw,), jnp.int32),
            pltpu.VMEM((bpw, _D), jnp.int32),
            pltpu.VMEM((bpw, _D * _NBITS), jnp.float32),
            pltpu.SemaphoreType.DMA,
        ],
        compiler_params=pltpu.CompilerParams(
            needs_layout_passes=False, use_tc_tiling_on_sc=False,
            skip_device_barrier=True),
    )
    def spike_embed(ids_hbm, table_hbm, out_hbm, idx_v, rows_v, out_v, sem):
        wid = lax.axis_index("s") * nc + lax.axis_index("c")
        base = wid * bpw
        pltpu.sync_copy(ids_hbm.at[pl.ds(base, bpw)], idx_v)
        pltpu.async_copy(table_hbm.at[idx_v], rows_v, sem).wait()

        lane = lax.iota(jnp.int32, nl)
        shifts = [31 - lane, 15 - lane]
        zeros = jnp.zeros((nl,), jnp.int32)

        def token_body(t, carry):
            row = rows_v[t]
            for d in range(_D):
                word = zeros + row[d]
                for h in range(2):
                    bits = ((word >> shifts[h]) & 1).astype(jnp.float32)
                    out_v[t, pl.ds(d * _NBITS + h * nl, nl)] = bits
            return carry

        lax.fori_loop(0, bpw, token_body, 0)
        pltpu.sync_copy(out_v, out_hbm.at[pl.ds(base, bpw)])

    return spike_embed(token_ids, weight_float)


def kernel(token_ids, weight_float):
    table_bits = jax.lax.bitcast_convert_type(
        weight_float.astype(jnp.float32), jnp.int32)
    out = _spike_embed_call(token_ids.astype(jnp.int32), table_bits)
    return out.reshape(_B, _D, _NBITS)
